# baseline (speedup 1.0000x reference)
import functools

import jax
import jax.numpy as jnp
from jax import lax
from jax.experimental import pallas as pl
from jax.experimental.pallas import tpu as pltpu
from jax.experimental.pallas import tpu_sc as plsc

L = 16
NUM_SPECIES = 16
TC_BLOCK = 1024

_GATHER_DNUMS = lax.GatherDimensionNumbers(
    offset_dims=(), collapsed_slice_dims=(0,), start_index_map=(0,))


def _vreg_gather(tab, idx):
    return lax.gather(
        tab, idx[:, None], _GATHER_DNUMS, slice_sizes=(1,),
        mode=lax.GatherScatterMode.PROMISE_IN_BOUNDS)


@functools.cache
def _build_sc(n_sc):
    info = plsc.get_sparse_core_info()
    nw = info.num_cores * info.num_subcores
    assert n_sc % (nw * L) == 0
    chunk = n_sc // nw

    mesh = plsc.VectorSubcoreMesh(core_axis_name="c", subcore_axis_name="s")

    @functools.partial(
        pl.kernel,
        mesh=mesh,
        out_type=jax.ShapeDtypeStruct((n_sc,), jnp.float32),
        scratch_types=[
            pltpu.VMEM((chunk,), jnp.int32),
            pltpu.VMEM((chunk,), jnp.float32),
            pltpu.VMEM((chunk,), jnp.float32),
            pltpu.VMEM((NUM_SPECIES,), jnp.float32),
            pltpu.VMEM((NUM_SPECIES,), jnp.float32),
            pltpu.SemaphoreType.DMA,
            pltpu.SemaphoreType.DMA,
        ],
    )
    def sc_kernel(x_hbm, t_hbm, shift_hbm, scale_hbm, out_hbm,
                  idx_v, x_v, out_v, shift_v, scale_v, sem_t, sem):
        wid = lax.axis_index("s") * info.num_cores + lax.axis_index("c")
        base = wid * chunk

        cps_t = [pltpu.async_copy(shift_hbm, shift_v, sem_t),
                 pltpu.async_copy(scale_hbm, scale_v, sem_t)]
        cps = [pltpu.async_copy(t_hbm.at[pl.ds(base, chunk)],
                                idx_v.at[pl.ds(0, chunk)], sem),
               pltpu.async_copy(x_hbm.at[pl.ds(base, chunk)],
                                x_v.at[pl.ds(0, chunk)], sem)]
        for cp in cps_t:
            cp.wait()
        s_tab = scale_v[...]
        b_tab = shift_v[...]
        for cp in cps:
            cp.wait()

        @plsc.parallel_loop(0, chunk // L, unroll=8)
        def body(i):
            o = i * L
            idx = idx_v[pl.ds(o, L)]
            xv = x_v[pl.ds(o, L)]
            s = _vreg_gather(s_tab, idx)
            b = _vreg_gather(b_tab, idx)
            out_v[pl.ds(o, L)] = xv * s + b

        pltpu.sync_copy(out_v.at[pl.ds(0, chunk)],
                        out_hbm.at[pl.ds(base, chunk)])

    return sc_kernel


def _tc_body(x_ref, t_ref, shift_ref, scale_ref, o_ref):
    xv = x_ref[...]
    tv = t_ref[...]
    s = jnp.zeros_like(xv)
    b = jnp.zeros_like(xv)
    for k in range(NUM_SPECIES):
        m = tv == k
        s = jnp.where(m, scale_ref[k], s)
        b = jnp.where(m, shift_ref[k], b)
    o_ref[...] = xv * s + b


@functools.cache
def _build_tc(n_blocks, first_block):
    return pl.pallas_call(
        _tc_body,
        grid=(n_blocks,),
        in_specs=[
            pl.BlockSpec((TC_BLOCK,), lambda i: (first_block + i,)),
            pl.BlockSpec((TC_BLOCK,), lambda i: (first_block + i,)),
            pl.BlockSpec(memory_space=pltpu.SMEM),
            pl.BlockSpec(memory_space=pltpu.SMEM),
        ],
        out_specs=pl.BlockSpec((TC_BLOCK,), lambda i: (i,)),
        out_shape=jax.ShapeDtypeStruct((n_blocks * TC_BLOCK,), jnp.float32),
    )


def kernel(scaled_atomic_energy, atom_type, shift, scale):
    n = scaled_atomic_energy.shape[0]
    x = scaled_atomic_energy.reshape(-1)
    t = atom_type.astype(jnp.int32)
    shift = shift.astype(jnp.float32)
    scale = scale.astype(jnp.float32)

    info = plsc.get_sparse_core_info()
    nw = info.num_cores * info.num_subcores
    n_sc = (2 * n // 3) // (nw * L) * (nw * L) // TC_BLOCK * TC_BLOCK
    first_block = n_sc // TC_BLOCK
    n_blocks = (n - n_sc + TC_BLOCK - 1) // TC_BLOCK

    out_sc = _build_sc(n_sc)(x, t, shift, scale)
    out_tc = _build_tc(n_blocks, first_block)(x, t, shift, scale)
    out = jnp.concatenate([out_sc, out_tc[: n - n_sc]])
    return out.reshape(n, 1)


# speedup vs baseline: 1.4951x; 1.4951x over previous
import functools

import jax
import jax.numpy as jnp
from jax import lax
from jax.experimental import pallas as pl
from jax.experimental.pallas import tpu as pltpu
from jax.experimental.pallas import tpu_sc as plsc

L = 16
NUM_SPECIES = 16

_GATHER_DNUMS = lax.GatherDimensionNumbers(
    offset_dims=(), collapsed_slice_dims=(0,), start_index_map=(0,))


def _vreg_gather(tab, idx):
    return lax.gather(
        tab, idx[:, None], _GATHER_DNUMS, slice_sizes=(1,),
        mode=lax.GatherScatterMode.PROMISE_IN_BOUNDS)


@functools.cache
def _build(n):
    info = plsc.get_sparse_core_info()
    nw = info.num_cores * info.num_subcores
    assert n % L == 0
    chunk = ((n + nw - 1) // nw + L - 1) // L * L
    chunk = (chunk + 2 * L - 1) // (2 * L) * (2 * L)
    assert chunk * (nw - 1) + chunk >= n and n - chunk >= 0
    half = chunk // 2

    mesh = plsc.VectorSubcoreMesh(core_axis_name="c", subcore_axis_name="s")

    @functools.partial(
        pl.kernel,
        mesh=mesh,
        out_type=jax.ShapeDtypeStruct((n,), jnp.float32),
        scratch_types=[
            pltpu.VMEM((chunk,), jnp.int32),
            pltpu.VMEM((chunk,), jnp.float32),
            pltpu.VMEM((chunk,), jnp.float32),
            pltpu.VMEM((NUM_SPECIES,), jnp.float32),
            pltpu.VMEM((NUM_SPECIES,), jnp.float32),
            pltpu.SemaphoreType.DMA,
            pltpu.SemaphoreType.DMA,
        ],
    )
    def sc_kernel(x_hbm, t_hbm, shift_hbm, scale_hbm, out_hbm,
                  idx_v, x_v, out_v, shift_v, scale_v, sem_t, sem):
        wid = lax.axis_index("s") * info.num_cores + lax.axis_index("c")
        base = jnp.minimum(wid * chunk, n - chunk)

        cps_t = [pltpu.async_copy(shift_hbm, shift_v, sem_t),
                 pltpu.async_copy(scale_hbm, scale_v, sem_t)]
        cps = [pltpu.async_copy(t_hbm.at[pl.ds(base, chunk)],
                                idx_v.at[pl.ds(0, chunk)], sem),
               pltpu.async_copy(x_hbm.at[pl.ds(base, chunk)],
                                x_v.at[pl.ds(0, chunk)], sem)]
        for cp in cps_t:
            cp.wait()
        s_tab = scale_v[...]
        b_tab = shift_v[...]
        for cp in cps:
            cp.wait()

        @plsc.parallel_loop(0, chunk // L, unroll=16)
        def body(i):
            o = i * L
            idx = idx_v[pl.ds(o, L)]
            xv = x_v[pl.ds(o, L)]
            s = _vreg_gather(s_tab, idx)
            b = _vreg_gather(b_tab, idx)
            out_v[pl.ds(o, L)] = xv * s + b

        pltpu.sync_copy(out_v.at[pl.ds(0, chunk)],
                        out_hbm.at[pl.ds(base, chunk)])

    return sc_kernel


def kernel(scaled_atomic_energy, atom_type, shift, scale):
    n = scaled_atomic_energy.shape[0]
    x = scaled_atomic_energy.reshape(-1)
    t = atom_type.astype(jnp.int32)
    out = _build(n)(x, t, shift.astype(jnp.float32), scale.astype(jnp.float32))
    return out.reshape(n, 1)


# speedup vs baseline: 1.5001x; 1.0033x over previous
import functools

import jax
import jax.numpy as jnp
from jax import lax
from jax.experimental import pallas as pl
from jax.experimental.pallas import tpu as pltpu
from jax.experimental.pallas import tpu_sc as plsc

L = 16
NUM_SPECIES = 16

_GATHER_DNUMS = lax.GatherDimensionNumbers(
    offset_dims=(), collapsed_slice_dims=(0,), start_index_map=(0,))


def _vreg_gather(tab, idx):
    return lax.gather(
        tab, idx[:, None], _GATHER_DNUMS, slice_sizes=(1,),
        mode=lax.GatherScatterMode.PROMISE_IN_BOUNDS)


@functools.cache
def _build(n):
    info = plsc.get_sparse_core_info()
    nw = info.num_cores * info.num_subcores
    assert n % L == 0
    chunk = ((n + nw - 1) // nw + L - 1) // L * L
    assert chunk * nw >= n and n - chunk >= 0

    mesh = plsc.VectorSubcoreMesh(core_axis_name="c", subcore_axis_name="s")

    @functools.partial(
        pl.kernel,
        mesh=mesh,
        out_type=jax.ShapeDtypeStruct((n,), jnp.float32),
        scratch_types=[
            pltpu.VMEM((chunk,), jnp.int32),
            pltpu.VMEM((chunk,), jnp.float32),
            pltpu.VMEM((chunk,), jnp.float32),
            pltpu.VMEM((NUM_SPECIES,), jnp.float32),
            pltpu.VMEM((NUM_SPECIES,), jnp.float32),
            pltpu.SemaphoreType.DMA,
            pltpu.SemaphoreType.DMA,
        ],
    )
    def sc_kernel(x_hbm, t_hbm, shift_hbm, scale_hbm, out_hbm,
                  idx_v, x_v, out_v, shift_v, scale_v, sem_t, sem):
        wid = lax.axis_index("s") * info.num_cores + lax.axis_index("c")
        base = jnp.minimum(wid * chunk, n - chunk)

        cps_t = [pltpu.async_copy(shift_hbm, shift_v, sem_t),
                 pltpu.async_copy(scale_hbm, scale_v, sem_t)]
        cps = [pltpu.async_copy(t_hbm.at[pl.ds(base, chunk)],
                                idx_v.at[pl.ds(0, chunk)], sem),
               pltpu.async_copy(x_hbm.at[pl.ds(base, chunk)],
                                x_v.at[pl.ds(0, chunk)], sem)]
        for cp in cps_t:
            cp.wait()
        s_tab = scale_v[...]
        b_tab = shift_v[...]
        for cp in cps:
            cp.wait()

        @plsc.parallel_loop(0, chunk // L, unroll=16)
        def body(i):
            o = i * L
            idx = idx_v[pl.ds(o, L)]
            xv = x_v[pl.ds(o, L)]
            s = _vreg_gather(s_tab, idx)
            b = _vreg_gather(b_tab, idx)
            out_v[pl.ds(o, L)] = xv * s + b

        pltpu.sync_copy(out_v.at[pl.ds(0, chunk)],
                        out_hbm.at[pl.ds(base, chunk)])

    return sc_kernel


def kernel(scaled_atomic_energy, atom_type, shift, scale):
    n = scaled_atomic_energy.shape[0]
    x = scaled_atomic_energy.reshape(-1)
    t = atom_type.astype(jnp.int32)
    out = _build(n)(x, t, shift.astype(jnp.float32), scale.astype(jnp.float32))
    return out.reshape(n, 1)
